# baseline (speedup 1.0000x reference)
import jax
import jax.numpy as jnp
from jax.experimental import pallas as pl
from jax.experimental.pallas import tpu as pltpu

_B = 64
_N = 3000
_NPAD = 3072
_JT = 768
_NJT = _NPAD // _JT
_GRP = 8
_NG = _B // _GRP
_SYM = (12, 15, 18, 19, 20)
_PAD_COORD = 1.0e6


def _loss_kernel(mask_ref, rt_ref, mpT_ref, tgtT_ref, out_ref):
    g = pl.program_id(0)

    col_id = jax.lax.broadcasted_iota(jnp.int32, (1, _NPAD), 1)
    col_ok = col_id < _N
    lane = jnp.ones((1, 128), jnp.float32)

    for s in range(_GRP):
        b = g * _GRP + s

        mx = mpT_ref[0, s, 0:1, :]
        my = mpT_ref[0, s, 1:2, :]
        mz = mpT_ref[0, s, 2:3, :]

        tfx = (mx * rt_ref[b, 0] + my * rt_ref[b, 3] + mz * rt_ref[b, 6]
               + rt_ref[b, 9])
        tfy = (mx * rt_ref[b, 1] + my * rt_ref[b, 4] + mz * rt_ref[b, 7]
               + rt_ref[b, 10])
        tfz = (mx * rt_ref[b, 2] + my * rt_ref[b, 5] + mz * rt_ref[b, 8]
               + rt_ref[b, 11])

        is_sym = mask_ref[b] != 0

        @pl.when(is_sym)
        def _sym_path(s=s, tfx=tfx, tfy=tfy, tfz=tfz):
            ones = jnp.ones_like(tfx)
            q2 = tfx * tfx + tfy * tfy + tfz * tfz
            yk = jnp.concatenate(
                [-2.0 * tfx, -2.0 * tfy, -2.0 * tfz, q2, ones], axis=0)
            gx = tgtT_ref[0, s, 0:1, :]
            gy = tgtT_ref[0, s, 1:2, :]
            gz = tgtT_ref[0, s, 2:3, :]
            r2 = gx * gx + gy * gy + gz * gz
            ones_r = jnp.ones_like(r2)
            dmin = None
            for j in range(_NJT):
                js = slice(j * _JT, (j + 1) * _JT)
                xkT = jnp.concatenate(
                    [tgtT_ref[0, s, :, js], ones_r[:, js], r2[:, js]],
                    axis=0)
                d2t = jax.lax.dot_general(
                    xkT, yk, (((0,), (0,)), ((), ())),
                    preferred_element_type=jnp.float32)
                tmin = jnp.min(d2t, axis=0, keepdims=True)
                dmin = tmin if dmin is None else jnp.minimum(dmin, tmin)
            dis = jnp.sqrt(jnp.maximum(dmin, 0.0))
            val = jnp.sum(jnp.where(col_ok, dis, 0.0)) / _N
            out_ref[0, s:s + 1, :] = val * lane

        @pl.when(jnp.logical_not(is_sym))
        def _direct_path(s=s, tfx=tfx, tfy=tfy, tfz=tfz):
            dx = tfx - tgtT_ref[0, s, 0:1, :]
            dy = tfy - tgtT_ref[0, s, 1:2, :]
            dz = tfz - tgtT_ref[0, s, 2:3, :]
            dis = jnp.sqrt(dx * dx + dy * dy + dz * dz)
            val = jnp.sum(jnp.where(col_ok, dis, 0.0)) / _N
            out_ref[0, s:s + 1, :] = val * lane


@jax.jit
def _run(mask, rt, mpT, tgtT):
    grid_spec = pltpu.PrefetchScalarGridSpec(
        num_scalar_prefetch=2,
        grid=(_NG,),
        in_specs=[
            pl.BlockSpec((1, _GRP, 3, _NPAD), lambda g, m, r: (g, 0, 0, 0)),
            pl.BlockSpec((1, _GRP, 3, _NPAD), lambda g, m, r: (g, 0, 0, 0)),
        ],
        out_specs=pl.BlockSpec((1, _GRP, 128), lambda g, m, r: (g, 0, 0)),
    )
    return pl.pallas_call(
        _loss_kernel,
        grid_spec=grid_spec,
        out_shape=jax.ShapeDtypeStruct((_NG, _GRP, 128), jnp.float32),
        compiler_params=pltpu.CompilerParams(
            dimension_semantics=("arbitrary",),
        ),
    )(mask, rt, mpT, tgtT)


def kernel(pred_r, pred_t, target, model_points, idx):
    pred_r = pred_r / jnp.linalg.norm(pred_r, axis=1, keepdims=True)
    w, x, y, z = pred_r[:, 0], pred_r[:, 1], pred_r[:, 2], pred_r[:, 3]
    rt = jnp.stack([
        1.0 - 2.0 * (y * y + z * z), 2.0 * (x * y - w * z), 2.0 * (x * z + w * y),
        2.0 * (x * y + w * z), 1.0 - 2.0 * (x * x + z * z), 2.0 * (y * z - w * x),
        2.0 * (x * z - w * y), 2.0 * (y * z + w * x), 1.0 - 2.0 * (x * x + y * y),
        pred_t[:, 0], pred_t[:, 1], pred_t[:, 2],
    ], axis=1)
    rt = jnp.pad(rt, ((0, 0), (0, 4)))

    sym = jnp.asarray(_SYM, dtype=idx.dtype)
    mask = (idx[:, 0][:, None] == sym[None, :]).any(axis=1).astype(jnp.int32)

    mp_pad = jnp.pad(model_points, ((0, 0), (0, _NPAD - _N), (0, 0)))
    t_pad = jnp.pad(target, ((0, 0), (0, _NPAD - _N), (0, 0)),
                    constant_values=_PAD_COORD)
    mpT = jnp.transpose(mp_pad, (0, 2, 1)).reshape(_NG, _GRP, 3, _NPAD)
    tgtT = jnp.transpose(t_pad, (0, 2, 1)).reshape(_NG, _GRP, 3, _NPAD)

    out = _run(mask, rt, mpT, tgtT)
    return out[:, :, 0].reshape(_B)


# speedup vs baseline: 3.4793x; 3.4793x over previous
import jax
import jax.numpy as jnp
from jax.experimental import pallas as pl
from jax.experimental.pallas import tpu as pltpu

_B = 64
_N = 3000
_NPAD = 3072
_JT = 768
_NJT = _NPAD // _JT
_SYM = (12, 15, 18, 19, 20)
_PAD_COORD = 1.0e6


def _loss_kernel(mask_ref, rt_ref, mpT_ref, tgtT_ref, out_ref):
    col_id = jax.lax.broadcasted_iota(jnp.int32, (1, _NPAD), 1)
    col_ok = col_id < _N
    lane = jnp.ones((1, 128), jnp.float32)

    def body(b, carry):
        mx = mpT_ref[b, 0:1, :]
        my = mpT_ref[b, 1:2, :]
        mz = mpT_ref[b, 2:3, :]

        tfx = (mx * rt_ref[b, 0] + my * rt_ref[b, 3] + mz * rt_ref[b, 6]
               + rt_ref[b, 9])
        tfy = (mx * rt_ref[b, 1] + my * rt_ref[b, 4] + mz * rt_ref[b, 7]
               + rt_ref[b, 10])
        tfz = (mx * rt_ref[b, 2] + my * rt_ref[b, 5] + mz * rt_ref[b, 8]
               + rt_ref[b, 11])

        is_sym = mask_ref[b] != 0

        @pl.when(is_sym)
        def _sym_path():
            ones = jnp.ones_like(tfx)
            q2 = tfx * tfx + tfy * tfy + tfz * tfz
            yk = jnp.concatenate(
                [-2.0 * tfx, -2.0 * tfy, -2.0 * tfz, q2, ones], axis=0)
            gx = tgtT_ref[b, 0:1, :]
            gy = tgtT_ref[b, 1:2, :]
            gz = tgtT_ref[b, 2:3, :]
            r2 = gx * gx + gy * gy + gz * gz
            ones_r = jnp.ones_like(r2)
            dmin = None
            for j in range(_NJT):
                js = slice(j * _JT, (j + 1) * _JT)
                xkT = jnp.concatenate(
                    [tgtT_ref[b, :, js], ones_r[:, js], r2[:, js]],
                    axis=0)
                d2t = jax.lax.dot_general(
                    xkT, yk, (((0,), (0,)), ((), ())),
                    preferred_element_type=jnp.float32)
                tmin = jnp.min(d2t, axis=0, keepdims=True)
                dmin = tmin if dmin is None else jnp.minimum(dmin, tmin)
            dis = jnp.sqrt(jnp.maximum(dmin, 0.0))
            val = jnp.sum(jnp.where(col_ok, dis, 0.0)) / _N
            out_ref[pl.ds(b, 1), :] = val * lane

        @pl.when(jnp.logical_not(is_sym))
        def _direct_path():
            dx = tfx - tgtT_ref[b, 0:1, :]
            dy = tfy - tgtT_ref[b, 1:2, :]
            dz = tfz - tgtT_ref[b, 2:3, :]
            dis = jnp.sqrt(dx * dx + dy * dy + dz * dz)
            val = jnp.sum(jnp.where(col_ok, dis, 0.0)) / _N
            out_ref[pl.ds(b, 1), :] = val * lane

        return carry

    jax.lax.fori_loop(0, _B, body, 0)


@jax.jit
def _run(mask, rt, mpT, tgtT):
    grid_spec = pltpu.PrefetchScalarGridSpec(
        num_scalar_prefetch=2,
        grid=(1,),
        in_specs=[
            pl.BlockSpec((_B, 3, _NPAD), lambda g, m, r: (0, 0, 0)),
            pl.BlockSpec((_B, 3, _NPAD), lambda g, m, r: (0, 0, 0)),
        ],
        out_specs=pl.BlockSpec((_B, 128), lambda g, m, r: (0, 0)),
    )
    return pl.pallas_call(
        _loss_kernel,
        grid_spec=grid_spec,
        out_shape=jax.ShapeDtypeStruct((_B, 128), jnp.float32),
        compiler_params=pltpu.CompilerParams(
            dimension_semantics=("arbitrary",),
        ),
    )(mask, rt, mpT, tgtT)


def kernel(pred_r, pred_t, target, model_points, idx):
    pred_r = pred_r / jnp.linalg.norm(pred_r, axis=1, keepdims=True)
    w, x, y, z = pred_r[:, 0], pred_r[:, 1], pred_r[:, 2], pred_r[:, 3]
    rt = jnp.stack([
        1.0 - 2.0 * (y * y + z * z), 2.0 * (x * y - w * z), 2.0 * (x * z + w * y),
        2.0 * (x * y + w * z), 1.0 - 2.0 * (x * x + z * z), 2.0 * (y * z - w * x),
        2.0 * (x * z - w * y), 2.0 * (y * z + w * x), 1.0 - 2.0 * (x * x + y * y),
        pred_t[:, 0], pred_t[:, 1], pred_t[:, 2],
    ], axis=1)
    rt = jnp.pad(rt, ((0, 0), (0, 4)))

    sym = jnp.asarray(_SYM, dtype=idx.dtype)
    mask = (idx[:, 0][:, None] == sym[None, :]).any(axis=1).astype(jnp.int32)

    mp_pad = jnp.pad(model_points, ((0, 0), (0, _NPAD - _N), (0, 0)))
    t_pad = jnp.pad(target, ((0, 0), (0, _NPAD - _N), (0, 0)),
                    constant_values=_PAD_COORD)
    mpT = jnp.transpose(mp_pad, (0, 2, 1))
    tgtT = jnp.transpose(t_pad, (0, 2, 1))

    out = _run(mask, rt, mpT, tgtT)
    return out[:, 0]


# speedup vs baseline: 4.0984x; 1.1779x over previous
import jax
import jax.numpy as jnp
from jax.experimental import pallas as pl
from jax.experimental.pallas import tpu as pltpu

_B = 64
_N = 3000
_NPAD = 3072
_JT = 768
_NJT = _NPAD // _JT
_SYM = (12, 15, 18, 19, 20)
_PAD_COORD = 1.0e6


def _loss_kernel(mask_ref, rtv_ref, mp3_ref, tgt3_ref, out_ref,
                 tfx_s, tfy_s, tfz_s, r2_s):
    col_id = jax.lax.broadcasted_iota(jnp.int32, (1, _NPAD), 1)
    col_ok = col_id < _N
    lane = jnp.ones((1, 128), jnp.float32)

    mpx = mp3_ref[0]
    mpy = mp3_ref[1]
    mpz = mp3_ref[2]

    def c(k):
        return rtv_ref[:, k:k + 1]

    tfx = mpx * c(0) + mpy * c(3) + mpz * c(6) + c(9)
    tfy = mpx * c(1) + mpy * c(4) + mpz * c(7) + c(10)
    tfz = mpx * c(2) + mpy * c(5) + mpz * c(8) + c(11)
    tfx_s[...] = tfx
    tfy_s[...] = tfy
    tfz_s[...] = tfz

    tgx = tgt3_ref[0]
    tgy = tgt3_ref[1]
    tgz = tgt3_ref[2]
    r2_s[...] = tgx * tgx + tgy * tgy + tgz * tgz

    dx = tfx - tgx
    dy = tfy - tgy
    dz = tfz - tgz
    dis = jnp.sqrt(dx * dx + dy * dy + dz * dz)
    dsum = jnp.sum(jnp.where(col_ok, dis, 0.0), axis=1, keepdims=True) / _N
    out_ref[...] = dsum * lane

    def body(b, carry):
        @pl.when(mask_ref[b] != 0)
        def _sym_path():
            bs = pl.ds(b, 1)
            tfxr = tfx_s[bs, :]
            tfyr = tfy_s[bs, :]
            tfzr = tfz_s[bs, :]
            ones = jnp.ones_like(tfxr)
            q2 = tfxr * tfxr + tfyr * tfyr + tfzr * tfzr
            yk = jnp.concatenate(
                [-2.0 * tfxr, -2.0 * tfyr, -2.0 * tfzr, q2, ones], axis=0)
            tgxr = tgt3_ref[0, bs, :]
            tgyr = tgt3_ref[1, bs, :]
            tgzr = tgt3_ref[2, bs, :]
            r2r = r2_s[bs, :]
            dmin = None
            for j in range(_NJT):
                js = slice(j * _JT, (j + 1) * _JT)
                xkT = jnp.concatenate(
                    [tgxr[:, js], tgyr[:, js], tgzr[:, js],
                     ones[:, js], r2r[:, js]], axis=0)
                d2t = jax.lax.dot_general(
                    xkT, yk, (((0,), (0,)), ((), ())),
                    preferred_element_type=jnp.float32)
                tmin = jnp.min(d2t, axis=0, keepdims=True)
                dmin = tmin if dmin is None else jnp.minimum(dmin, tmin)
            disr = jnp.sqrt(jnp.maximum(dmin, 0.0))
            val = jnp.sum(jnp.where(col_ok, disr, 0.0)) / _N
            out_ref[bs, :] = val * lane

        return carry

    jax.lax.fori_loop(0, _B, body, 0)


@jax.jit
def _run(mask, rtv, mp3, tgt3):
    grid_spec = pltpu.PrefetchScalarGridSpec(
        num_scalar_prefetch=1,
        grid=(1,),
        in_specs=[
            pl.BlockSpec((_B, 16), lambda g, m: (0, 0)),
            pl.BlockSpec((3, _B, _NPAD), lambda g, m: (0, 0, 0)),
            pl.BlockSpec((3, _B, _NPAD), lambda g, m: (0, 0, 0)),
        ],
        out_specs=pl.BlockSpec((_B, 128), lambda g, m: (0, 0)),
        scratch_shapes=[pltpu.VMEM((_B, _NPAD), jnp.float32)] * 4,
    )
    return pl.pallas_call(
        _loss_kernel,
        grid_spec=grid_spec,
        out_shape=jax.ShapeDtypeStruct((_B, 128), jnp.float32),
        compiler_params=pltpu.CompilerParams(
            dimension_semantics=("arbitrary",),
        ),
    )(mask, rtv, mp3, tgt3)


def kernel(pred_r, pred_t, target, model_points, idx):
    pred_r = pred_r / jnp.linalg.norm(pred_r, axis=1, keepdims=True)
    w, x, y, z = pred_r[:, 0], pred_r[:, 1], pred_r[:, 2], pred_r[:, 3]
    rt = jnp.stack([
        1.0 - 2.0 * (y * y + z * z), 2.0 * (x * y - w * z), 2.0 * (x * z + w * y),
        2.0 * (x * y + w * z), 1.0 - 2.0 * (x * x + z * z), 2.0 * (y * z - w * x),
        2.0 * (x * z - w * y), 2.0 * (y * z + w * x), 1.0 - 2.0 * (x * x + y * y),
        pred_t[:, 0], pred_t[:, 1], pred_t[:, 2],
    ], axis=1)
    rtv = jnp.pad(rt, ((0, 0), (0, 4)))

    sym = jnp.asarray(_SYM, dtype=idx.dtype)
    mask = (idx[:, 0][:, None] == sym[None, :]).any(axis=1).astype(jnp.int32)

    mp_pad = jnp.pad(model_points, ((0, 0), (0, _NPAD - _N), (0, 0)))
    t_pad = jnp.pad(target, ((0, 0), (0, _NPAD - _N), (0, 0)),
                    constant_values=_PAD_COORD)
    mp3 = jnp.transpose(mp_pad, (2, 0, 1))
    tgt3 = jnp.transpose(t_pad, (2, 0, 1))

    out = _run(mask, rtv, mp3, tgt3)
    return out[:, 0]


# speedup vs baseline: 4.1023x; 1.0010x over previous
import jax
import jax.numpy as jnp
from jax.experimental import pallas as pl
from jax.experimental.pallas import tpu as pltpu

_B = 64
_N = 3000
_NPAD = 3072
_JT = 1536
_NJT = _NPAD // _JT
_SYM = (12, 15, 18, 19, 20)
_PAD_COORD = 1.0e6


def _loss_kernel(mask_ref, rtv_ref, mp3_ref, tgt3_ref, out_ref,
                 tfx_s, tfy_s, tfz_s, r2_s):
    col_id = jax.lax.broadcasted_iota(jnp.int32, (1, _NPAD), 1)
    col_ok = col_id < _N
    lane = jnp.ones((1, 128), jnp.float32)

    mpx = mp3_ref[0]
    mpy = mp3_ref[1]
    mpz = mp3_ref[2]

    def c(k):
        return rtv_ref[:, k:k + 1]

    tfx = mpx * c(0) + mpy * c(3) + mpz * c(6) + c(9)
    tfy = mpx * c(1) + mpy * c(4) + mpz * c(7) + c(10)
    tfz = mpx * c(2) + mpy * c(5) + mpz * c(8) + c(11)
    tfx_s[...] = tfx
    tfy_s[...] = tfy
    tfz_s[...] = tfz

    tgx = tgt3_ref[0]
    tgy = tgt3_ref[1]
    tgz = tgt3_ref[2]
    r2_s[...] = tgx * tgx + tgy * tgy + tgz * tgz

    dx = tfx - tgx
    dy = tfy - tgy
    dz = tfz - tgz
    dis = jnp.sqrt(dx * dx + dy * dy + dz * dz)
    dsum = jnp.sum(jnp.where(col_ok, dis, 0.0), axis=1, keepdims=True) / _N
    out_ref[...] = dsum * lane

    def body(b, carry):
        @pl.when(mask_ref[b] != 0)
        def _sym_path():
            bs = pl.ds(b, 1)
            tfxr = tfx_s[bs, :]
            tfyr = tfy_s[bs, :]
            tfzr = tfz_s[bs, :]
            ones = jnp.ones_like(tfxr)
            q2 = tfxr * tfxr + tfyr * tfyr + tfzr * tfzr
            yk = jnp.concatenate(
                [-2.0 * tfxr, -2.0 * tfyr, -2.0 * tfzr, q2, ones], axis=0)
            tgxr = tgt3_ref[0, bs, :]
            tgyr = tgt3_ref[1, bs, :]
            tgzr = tgt3_ref[2, bs, :]
            r2r = r2_s[bs, :]
            dmin = None
            for j in range(_NJT):
                js = slice(j * _JT, (j + 1) * _JT)
                xkT = jnp.concatenate(
                    [tgxr[:, js], tgyr[:, js], tgzr[:, js],
                     ones[:, js], r2r[:, js]], axis=0)
                d2t = jax.lax.dot_general(
                    xkT, yk, (((0,), (0,)), ((), ())),
                    preferred_element_type=jnp.float32)
                tmin = jnp.min(d2t, axis=0, keepdims=True)
                dmin = tmin if dmin is None else jnp.minimum(dmin, tmin)
            disr = jnp.sqrt(jnp.maximum(dmin, 0.0))
            val = jnp.sum(jnp.where(col_ok, disr, 0.0)) / _N
            out_ref[bs, :] = val * lane

        return carry

    jax.lax.fori_loop(0, _B, body, 0)


@jax.jit
def _run(mask, rtv, mp3, tgt3):
    grid_spec = pltpu.PrefetchScalarGridSpec(
        num_scalar_prefetch=1,
        grid=(1,),
        in_specs=[
            pl.BlockSpec((_B, 16), lambda g, m: (0, 0)),
            pl.BlockSpec((3, _B, _NPAD), lambda g, m: (0, 0, 0)),
            pl.BlockSpec((3, _B, _NPAD), lambda g, m: (0, 0, 0)),
        ],
        out_specs=pl.BlockSpec((_B, 128), lambda g, m: (0, 0)),
        scratch_shapes=[pltpu.VMEM((_B, _NPAD), jnp.float32)] * 4,
    )
    return pl.pallas_call(
        _loss_kernel,
        grid_spec=grid_spec,
        out_shape=jax.ShapeDtypeStruct((_B, 128), jnp.float32),
        compiler_params=pltpu.CompilerParams(
            dimension_semantics=("arbitrary",),
        ),
    )(mask, rtv, mp3, tgt3)


def kernel(pred_r, pred_t, target, model_points, idx):
    pred_r = pred_r / jnp.linalg.norm(pred_r, axis=1, keepdims=True)
    w, x, y, z = pred_r[:, 0], pred_r[:, 1], pred_r[:, 2], pred_r[:, 3]
    rt = jnp.stack([
        1.0 - 2.0 * (y * y + z * z), 2.0 * (x * y - w * z), 2.0 * (x * z + w * y),
        2.0 * (x * y + w * z), 1.0 - 2.0 * (x * x + z * z), 2.0 * (y * z - w * x),
        2.0 * (x * z - w * y), 2.0 * (y * z + w * x), 1.0 - 2.0 * (x * x + y * y),
        pred_t[:, 0], pred_t[:, 1], pred_t[:, 2],
    ], axis=1)
    rtv = jnp.pad(rt, ((0, 0), (0, 4)))

    sym = jnp.asarray(_SYM, dtype=idx.dtype)
    mask = (idx[:, 0][:, None] == sym[None, :]).any(axis=1).astype(jnp.int32)

    mp_pad = jnp.pad(model_points, ((0, 0), (0, _NPAD - _N), (0, 0)))
    t_pad = jnp.pad(target, ((0, 0), (0, _NPAD - _N), (0, 0)),
                    constant_values=_PAD_COORD)
    mp3 = jnp.transpose(mp_pad, (2, 0, 1))
    tgt3 = jnp.transpose(t_pad, (2, 0, 1))

    out = _run(mask, rtv, mp3, tgt3)
    return out[:, 0]
